# baseline (speedup 1.0000x reference)
import jax
import jax.numpy as jnp
from jax import lax
from jax.experimental import pallas as pl
from jax.experimental.pallas import tpu as pltpu
from jax.experimental.pallas import tpu_sc as plsc

N, E, D = 10000, 160000, 256
HALF = D // 2
BN_EPS = 1e-5
NS = 16
CH = 80
EPT = E // NS
NCHUNK = EPT // CH
RING = 4
DRING = 2
NGRP = (NCHUNK - 1) // RING
ZROWS = 624
ZTAIL = N - NS * ZROWS
LANES = 16


def _sc_agg_body(*refs):
    (xs_hbm, src_hbm, dst_hbm, ea_hbm, zeros_hbm, agg_hbm) = refs[:6]
    r = refs[6:]
    idxs_v = r[0:RING]
    idxd_v = r[RING:2 * RING]
    xg_v = r[2 * RING:2 * RING + DRING]
    ea_v = r[2 * RING + DRING:2 * RING + 2 * DRING]
    agg_sh = r[2 * RING + 2 * DRING]
    sems = r[2 * RING + 2 * DRING + 1:]
    issem = sems[0:RING]
    idsem = sems[RING:2 * RING]
    gsem = sems[2 * RING:2 * RING + DRING]
    esem = sems[2 * RING + DRING:2 * RING + 2 * DRING]

    c = lax.axis_index("c")
    s = lax.axis_index("s")

    pltpu.sync_copy(zeros_hbm, agg_sh.at[pl.ds(s * ZROWS, ZROWS)])

    @pl.when(s == NS - 1)
    def _zero_tail():
        pltpu.sync_copy(zeros_hbm.at[pl.ds(0, ZTAIL)],
                        agg_sh.at[pl.ds(NS * ZROWS, ZTAIL)])

    plsc.subcore_barrier()

    ebase = s * EPT
    ccol = c * HALF
    sbase0 = c * E + ebase

    def issue_idx(t, slot):
        base = ebase + t * CH
        pltpu.async_copy(src_hbm.at[pl.ds(sbase0 + t * CH, CH)], idxs_v[slot],
                         issem[slot])
        pltpu.async_copy(dst_hbm.at[pl.ds(base, CH)], idxd_v[slot], idsem[slot])

    def wait_idx(t, slot):
        base = ebase + t * CH
        pltpu.make_async_copy(src_hbm.at[pl.ds(sbase0 + t * CH, CH)],
                              idxs_v[slot], issem[slot]).wait()
        pltpu.make_async_copy(dst_hbm.at[pl.ds(base, CH)], idxd_v[slot],
                              idsem[slot]).wait()

    def issue_loads(t, islot, dslot):
        base = ebase + t * CH
        pltpu.async_copy(xs_hbm.at[idxs_v[islot]], xg_v[dslot], gsem[dslot])
        pltpu.async_copy(ea_hbm.at[pl.ds(base, CH), pl.ds(ccol, HALF)],
                         ea_v[dslot], esem[dslot])

    def wait_loads(t, islot, dslot):
        base = ebase + t * CH
        pltpu.make_async_copy(xs_hbm.at[idxs_v[islot]], xg_v[dslot],
                              gsem[dslot]).wait()
        pltpu.make_async_copy(ea_hbm.at[pl.ds(base, CH), pl.ds(ccol, HALF)],
                              ea_v[dslot], esem[dslot]).wait()

    def scatter(islot, dslot):
        pltpu.sync_copy(xg_v[dslot], agg_sh.at[idxd_v[islot]], add=True)

    RU = 4

    def compute(dslot):
        def _rows(rq, carry):
            for rj in range(RU):
                rr = rq * RU + rj
                for k in range(HALF // LANES):
                    sl = pl.ds(k * LANES, LANES)
                    xg_v[dslot][rr, sl] = jnp.maximum(
                        xg_v[dslot][rr, sl] + ea_v[dslot][rr, sl], 0.0)
            return carry

        lax.fori_loop(0, CH // RU, _rows, 0)

    issue_idx(0, 0)
    issue_idx(1, 1)
    issue_idx(2, 2)
    wait_idx(0, 0)
    issue_loads(0, 0, 0)

    def group_body(g, carry):
        t0 = g * RING
        for j in range(RING):
            t = t0 + j
            wait_idx(t + 1, (j + 1) % RING)
            issue_loads(t + 1, (j + 1) % RING, (j + 1) % DRING)

            @pl.when(t + 3 < NCHUNK)
            def _prefetch_idx():
                issue_idx(t + 3, (j + 3) % RING)

            wait_loads(t, j, j % DRING)
            compute(j % DRING)
            scatter(j, j % DRING)
        return carry

    lax.fori_loop(0, NGRP, group_body, 0)

    wait_loads(NCHUNK - 1, (NCHUNK - 1) % RING, (NCHUNK - 1) % DRING)
    compute((NCHUNK - 1) % DRING)
    scatter((NCHUNK - 1) % RING, (NCHUNK - 1) % DRING)

    plsc.subcore_barrier()
    r0 = s * ZROWS
    pltpu.sync_copy(agg_sh.at[pl.ds(r0, ZROWS)],
                    agg_hbm.at[pl.ds(r0, ZROWS), pl.ds(ccol, HALF)])

    @pl.when(s == NS - 1)
    def _drain_tail():
        pltpu.sync_copy(agg_sh.at[pl.ds(NS * ZROWS, ZTAIL)],
                        agg_hbm.at[pl.ds(NS * ZROWS, ZTAIL), pl.ds(ccol, HALF)])


def _sc_aggregate(xs, src, dst, edge_attr, zeros):
    mesh = plsc.VectorSubcoreMesh(core_axis_name="c", subcore_axis_name="s")
    scratch = (
        [pltpu.VMEM((CH,), jnp.int32) for _ in range(2 * RING)]
        + [pltpu.VMEM((CH, HALF), jnp.float32) for _ in range(2 * DRING)]
        + [pltpu.VMEM_SHARED((N, HALF), jnp.float32)]
        + [pltpu.SemaphoreType.DMA for _ in range(2 * RING + 2 * DRING)]
    )
    return pl.kernel(
        _sc_agg_body,
        out_type=jax.ShapeDtypeStruct((N, D), jnp.float32),
        mesh=mesh,
        scratch_types=scratch,
    )(xs, src, dst, edge_attr, zeros)


BLK = 1000


def _mlp_body(ope_ref, x_ref, agg_ref, w1_ref, b1_ref, w2_ref, b2_ref, o_ref):
    h0 = x_ref[...] * ope_ref[0, 0] + agg_ref[...]
    h1 = jnp.dot(h0, w1_ref[...], preferred_element_type=jnp.float32)
    h1 = jnp.maximum(h1 + b1_ref[...], 0.0)
    h2 = jnp.dot(h1, w2_ref[...], preferred_element_type=jnp.float32)
    o_ref[...] = jnp.maximum(h2 + b2_ref[...], 0.0)


def _mlp(ope, x, agg, w1f, b1f, w2f, b2f):
    return pl.pallas_call(
        _mlp_body,
        grid=(N // BLK,),
        in_specs=[
            pl.BlockSpec(memory_space=pltpu.SMEM),
            pl.BlockSpec((BLK, D), lambda i: (i, 0)),
            pl.BlockSpec((BLK, D), lambda i: (i, 0)),
            pl.BlockSpec((D, 2 * D), lambda i: (0, 0)),
            pl.BlockSpec((1, 2 * D), lambda i: (0, 0)),
            pl.BlockSpec((2 * D, D), lambda i: (0, 0)),
            pl.BlockSpec((1, D), lambda i: (0, 0)),
        ],
        out_specs=pl.BlockSpec((BLK, D), lambda i: (i, 0)),
        out_shape=jax.ShapeDtypeStruct((N, D), jnp.float32),
    )(ope, x, agg, w1f, b1f, w2f, b2f)


def kernel(x, edge_index, edge_attr, W1, b1, g1, beta1, W2, b2, g2, beta2, g3, beta3, eps):
    src = edge_index[0]
    dst = edge_index[1]
    xs = jnp.concatenate([x[:, :HALF], x[:, HALF:]], axis=0)
    src2 = jnp.concatenate([src, src + N])
    zeros = jnp.zeros((ZROWS, HALF), jnp.float32)
    agg = _sc_aggregate(xs, src2, dst, edge_attr, zeros)

    inv = 1.0 / jnp.sqrt(jnp.float32(1.0 + BN_EPS))
    s1 = g1 * inv
    w1f = (W1 * s1[:, None]).T
    b1f = (b1 * s1 + beta1).reshape(1, 2 * D)
    s2 = g2 * inv
    w2f = (W2 * s2[:, None]).T
    b2f = (b2 * s2 + beta2).reshape(1, D)
    ope = (1.0 + eps).reshape(1, 1)
    return _mlp(ope, x, agg, w1f, b1f, w2f, b2f)


# speedup vs baseline: 1.0221x; 1.0221x over previous
import jax
import jax.numpy as jnp
from jax import lax
from jax.experimental import pallas as pl
from jax.experimental.pallas import tpu as pltpu
from jax.experimental.pallas import tpu_sc as plsc

N, E, D = 10000, 160000, 256
HALF = D // 2
BN_EPS = 1e-5
NS = 16
CH = 80
EPT = E // NS
NCHUNK = EPT // CH
RING = 4
DRING = 2
NGRP = (NCHUNK - 1) // RING
ZROWS = 624
ZTAIL = N - NS * ZROWS
LANES = 16


def _sc_agg_body(*refs):
    (xs_hbm, src_hbm, dst_hbm, ea_hbm, zeros_hbm, agg_hbm) = refs[:6]
    r = refs[6:]
    idxs_v = r[0:RING]
    idxd_v = r[RING:2 * RING]
    xg_v = r[2 * RING:2 * RING + DRING]
    ea_v = r[2 * RING + DRING:2 * RING + 2 * DRING]
    agg_sh = r[2 * RING + 2 * DRING]
    sems = r[2 * RING + 2 * DRING + 1:]
    issem = sems[0:RING]
    idsem = sems[RING:2 * RING]
    gsem = sems[2 * RING:2 * RING + DRING]
    esem = sems[2 * RING + DRING:2 * RING + 2 * DRING]

    c = lax.axis_index("c")
    s = lax.axis_index("s")

    pltpu.sync_copy(zeros_hbm, agg_sh.at[pl.ds(s * ZROWS, ZROWS)])

    @pl.when(s == NS - 1)
    def _zero_tail():
        pltpu.sync_copy(zeros_hbm.at[pl.ds(0, ZTAIL)],
                        agg_sh.at[pl.ds(NS * ZROWS, ZTAIL)])

    plsc.subcore_barrier()

    ebase = s * EPT
    ccol = c * HALF

    def issue_idx(t, slot):
        base = ebase + t * CH
        pltpu.async_copy(src_hbm.at[pl.ds(base, CH)], idxs_v[slot], issem[slot])
        pltpu.async_copy(dst_hbm.at[pl.ds(base, CH)], idxd_v[slot], idsem[slot])

    def wait_idx(t, slot):
        base = ebase + t * CH
        pltpu.make_async_copy(src_hbm.at[pl.ds(base, CH)], idxs_v[slot],
                              issem[slot]).wait()
        pltpu.make_async_copy(dst_hbm.at[pl.ds(base, CH)], idxd_v[slot],
                              idsem[slot]).wait()

    def issue_loads(t, islot, dslot):
        base = ebase + t * CH
        pltpu.async_copy(xs_hbm.at[idxs_v[islot], pl.ds(ccol, HALF)],
                         xg_v[dslot], gsem[dslot])
        pltpu.async_copy(ea_hbm.at[pl.ds(base, CH), pl.ds(ccol, HALF)],
                         ea_v[dslot], esem[dslot])

    def wait_loads(t, islot, dslot):
        base = ebase + t * CH
        pltpu.make_async_copy(xs_hbm.at[idxs_v[islot], pl.ds(ccol, HALF)],
                              xg_v[dslot], gsem[dslot]).wait()
        pltpu.make_async_copy(ea_hbm.at[pl.ds(base, CH), pl.ds(ccol, HALF)],
                              ea_v[dslot], esem[dslot]).wait()

    def scatter(islot, dslot):
        pltpu.sync_copy(xg_v[dslot], agg_sh.at[idxd_v[islot]], add=True)

    RU = 4

    def compute(dslot):
        def _rows(rq, carry):
            for rj in range(RU):
                rr = rq * RU + rj
                for k in range(HALF // LANES):
                    sl = pl.ds(k * LANES, LANES)
                    xg_v[dslot][rr, sl] = jnp.maximum(
                        xg_v[dslot][rr, sl] + ea_v[dslot][rr, sl], 0.0)
            return carry

        lax.fori_loop(0, CH // RU, _rows, 0)

    issue_idx(0, 0)
    issue_idx(1, 1)
    issue_idx(2, 2)
    wait_idx(0, 0)
    issue_loads(0, 0, 0)

    def group_body(g, carry):
        t0 = g * RING
        for j in range(RING):
            t = t0 + j
            wait_idx(t + 1, (j + 1) % RING)
            issue_loads(t + 1, (j + 1) % RING, (j + 1) % DRING)

            @pl.when(t + 3 < NCHUNK)
            def _prefetch_idx():
                issue_idx(t + 3, (j + 3) % RING)

            wait_loads(t, j, j % DRING)
            compute(j % DRING)
            scatter(j, j % DRING)
        return carry

    lax.fori_loop(0, NGRP, group_body, 0)

    wait_loads(NCHUNK - 1, (NCHUNK - 1) % RING, (NCHUNK - 1) % DRING)
    compute((NCHUNK - 1) % DRING)
    scatter((NCHUNK - 1) % RING, (NCHUNK - 1) % DRING)

    plsc.subcore_barrier()
    r0 = s * ZROWS
    pltpu.sync_copy(agg_sh.at[pl.ds(r0, ZROWS)],
                    agg_hbm.at[pl.ds(r0, ZROWS), pl.ds(ccol, HALF)])

    @pl.when(s == NS - 1)
    def _drain_tail():
        pltpu.sync_copy(agg_sh.at[pl.ds(NS * ZROWS, ZTAIL)],
                        agg_hbm.at[pl.ds(NS * ZROWS, ZTAIL), pl.ds(ccol, HALF)])


def _sc_aggregate(xs, src, dst, edge_attr, zeros):
    mesh = plsc.VectorSubcoreMesh(core_axis_name="c", subcore_axis_name="s")
    scratch = (
        [pltpu.VMEM((CH,), jnp.int32) for _ in range(2 * RING)]
        + [pltpu.VMEM((CH, HALF), jnp.float32) for _ in range(2 * DRING)]
        + [pltpu.VMEM_SHARED((N, HALF), jnp.float32)]
        + [pltpu.SemaphoreType.DMA for _ in range(2 * RING + 2 * DRING)]
    )
    return pl.kernel(
        _sc_agg_body,
        out_type=jax.ShapeDtypeStruct((N, D), jnp.float32),
        mesh=mesh,
        scratch_types=scratch,
    )(xs, src, dst, edge_attr, zeros)


BLK = 1000


def _mlp_body(ope_ref, x_ref, agg_ref, w1_ref, b1_ref, w2_ref, b2_ref, o_ref):
    h0 = x_ref[...] * ope_ref[0, 0] + agg_ref[...]
    h1 = jnp.dot(h0, w1_ref[...], preferred_element_type=jnp.float32)
    h1 = jnp.maximum(h1 + b1_ref[...], 0.0)
    h2 = jnp.dot(h1, w2_ref[...], preferred_element_type=jnp.float32)
    o_ref[...] = jnp.maximum(h2 + b2_ref[...], 0.0)


def _mlp(ope, x, agg, w1f, b1f, w2f, b2f):
    return pl.pallas_call(
        _mlp_body,
        grid=(N // BLK,),
        in_specs=[
            pl.BlockSpec(memory_space=pltpu.SMEM),
            pl.BlockSpec((BLK, D), lambda i: (i, 0)),
            pl.BlockSpec((BLK, D), lambda i: (i, 0)),
            pl.BlockSpec((D, 2 * D), lambda i: (0, 0)),
            pl.BlockSpec((1, 2 * D), lambda i: (0, 0)),
            pl.BlockSpec((2 * D, D), lambda i: (0, 0)),
            pl.BlockSpec((1, D), lambda i: (0, 0)),
        ],
        out_specs=pl.BlockSpec((BLK, D), lambda i: (i, 0)),
        out_shape=jax.ShapeDtypeStruct((N, D), jnp.float32),
    )(ope, x, agg, w1f, b1f, w2f, b2f)


def kernel(x, edge_index, edge_attr, W1, b1, g1, beta1, W2, b2, g2, beta2, g3, beta3, eps):
    src = edge_index[0]
    dst = edge_index[1]
    zeros = jnp.zeros((ZROWS, HALF), jnp.float32)
    agg = _sc_aggregate(x, src, dst, edge_attr, zeros)

    inv = 1.0 / jnp.sqrt(jnp.float32(1.0 + BN_EPS))
    s1 = g1 * inv
    w1f = (W1 * s1[:, None]).T
    b1f = (b1 * s1 + beta1).reshape(1, 2 * D)
    s2 = g2 * inv
    w2f = (W2 * s2[:, None]).T
    b2f = (b2 * s2 + beta2).reshape(1, D)
    ope = (1.0 + eps).reshape(1, 1)
    return _mlp(ope, x, agg, w1f, b1f, w2f, b2f)
